# baseline (speedup 1.0000x reference)
import functools

import jax
import jax.numpy as jnp
from jax import lax
from jax.experimental import pallas as pl
from jax.experimental.pallas import tpu as pltpu
from jax.experimental.pallas import tpu_sc as plsc

_LANES = 16
_CHUNK = 128
_ROW = 128


def _log2(x):
    assert x & (x - 1) == 0 and x > 0
    return x.bit_length() - 1


@functools.lru_cache(maxsize=None)
def _build(B, HW, C, NPAD):
    n_workers = 32
    per_w = (B * NPAD) // n_workers
    n_chunks = per_w // _CHUNK
    workers_per_batch = n_workers // B
    rows_per_big = _ROW // C
    rows_per_vec = _LANES // C
    n_vecs = per_w // rows_per_vec
    vecs_per_chunk = n_vecs // n_chunks
    rpb_sh, c_sh, chunk_sh = (jnp.int32(_log2(rows_per_big)),
                              jnp.int32(_log2(C)), jnp.int32(_log2(_CHUNK)))

    mesh = plsc.VectorSubcoreMesh(core_axis_name="c", subcore_axis_name="s")

    @functools.partial(
        pl.kernel,
        mesh=mesh,
        compiler_params=pltpu.CompilerParams(needs_layout_passes=False),
        out_type=jax.ShapeDtypeStruct((B * NPAD * C // _LANES, _LANES),
                                      jnp.float32),
        scratch_types=[
            pltpu.VMEM((n_chunks, _CHUNK), jnp.int32),
            pltpu.VMEM((n_chunks, _CHUNK), jnp.int32),
            pltpu.VMEM((n_chunks, _CHUNK, _ROW), jnp.float32),
            pltpu.VMEM((n_vecs, _LANES), jnp.float32),
            pltpu.SemaphoreType.DMA,
        ],
    )
    def gather(table_hbm, idx_hbm, out_hbm, idx_v, gid_v, rows_v, out_v, sem):
        wid = lax.axis_index("s") * 2 + lax.axis_index("c")
        pltpu.sync_copy(idx_hbm.at[pl.ds(wid * n_chunks, n_chunks)], idx_v)
        big_off = (wid // workers_per_batch) * (HW * C // _ROW)
        for j in range(n_chunks):
            for k in range(_CHUNK // _LANES):
                sl = pl.ds(k * _LANES, _LANES)
                gid_v[j, sl] = lax.shift_right_logical(
                    idx_v[j, sl], rpb_sh) + big_off
        copies = [
            pltpu.async_copy(table_hbm.at[gid_v.at[j]], rows_v.at[j], sem)
            for j in range(n_chunks)
        ]
        lane = lax.iota(jnp.int32, _LANES)
        lane_row = lax.shift_right_logical(lane, c_sh)
        lane_elem = lane & (C - 1)
        for j, cp in enumerate(copies):
            cp.wait()
            for t in range(j * vecs_per_chunk, (j + 1) * vecs_per_chunk):
                rid = t * rows_per_vec + lane_row
                raw = plsc.load_gather(
                    idx_v, [lax.shift_right_logical(rid, chunk_sh),
                            rid & (_CHUNK - 1)])
                sub = lax.shift_left((raw & (rows_per_big - 1)), c_sh)
                val = plsc.load_gather(
                    rows_v, [lax.shift_right_logical(rid, chunk_sh),
                             rid & (_CHUNK - 1), sub + lane_elem])
                out_v[t] = val
        pltpu.sync_copy(out_v, out_hbm.at[pl.ds(wid * n_vecs, n_vecs)])

    return gather


def kernel(feat, index):
    B, H, W, C = feat.shape
    N = index.shape[1]
    HW = H * W
    group = (32 // B) * _CHUNK if B <= 32 else _CHUNK
    NPAD = ((N + group - 1) // group) * group
    idx = jnp.pad(index, ((0, 0), (0, NPAD - N)))
    idx = idx.reshape((B * NPAD) // _CHUNK, _CHUNK)
    table = feat.reshape(B * HW * C // _ROW, _ROW)
    out = _build(B, HW, C, NPAD)(table, idx)
    return out.reshape(B, NPAD, C)[:, :N, :]
